# Initial kernel scaffold; baseline (speedup 1.0000x reference)
import jax
import jax.numpy as jnp
from jax.experimental import pallas as pl


def kernel(llr, pcm, s, bs):
    raise NotImplementedError("write your pallas kernel here")


# speedup vs baseline: 45.4661x; 45.4661x over previous
import functools

import jax
import jax.numpy as jnp
from jax import lax
from jax.experimental import pallas as pl
from jax.experimental.pallas import tpu as pltpu
from jax.experimental.pallas import tpu_sc as plsc

L = 16
W = 16
NWORDS = W + 1
RANKR = 256
NCOL = 512
BIG = 1 << 20


def _build_elim(num_cores, num_subcores):
    nworkers = num_cores * num_subcores
    assert 64 % nworkers == 0
    bpw = 64 // nworkers

    mesh = plsc.VectorSubcoreMesh(core_axis_name="c", subcore_axis_name="s")

    def body(words_hbm, rankt_hbm, so_hbm, idx_hbm, sol_hbm, M, rankT, so_v, idxv):
        wid = lax.axis_index("s") * num_cores + lax.axis_index("c")
        iota = lax.iota(jnp.int32, L)

        def lane(vec, i):
            return jnp.sum(jnp.where(iota == i, vec, 0))

        for t in range(bpw):
            b = wid * bpw + t
            pltpu.sync_copy(words_hbm.at[b], M)
            pltpu.sync_copy(rankt_hbm.at[b], rankT)
            pltpu.sync_copy(so_hbm.at[b], so_v)

            def step(r, idx_vec):
                roww = plsc.load_gather(M, [iota, jnp.full((L,), r, jnp.int32)])

                def pf(j, best):
                    bit = lax.shift_right_logical(
                        roww, jnp.full((L,), j, jnp.int32)) & 1
                    cand = jnp.where(bit == 1, rankT[j], BIG)
                    return jnp.minimum(best, cand)

                best = lax.fori_loop(0, 32, pf, jnp.full((L,), BIG, jnp.int32))
                bmin = jnp.min(best)
                sw = lane(M[W, pl.ds((r >> 4) * L, L)], r & (L - 1))
                bmin = jnp.where(sw != 0, jnp.minimum(bmin, NCOL), bmin)

                piv = jnp.where(bmin >= BIG, 0, bmin).astype(jnp.int32)
                idx_vec = jnp.where(iota == (r & (L - 1)), piv, idx_vec)

                @pl.when((r & (L - 1)) == (L - 1))
                def _store_idx():
                    idxv[pl.ds((r >> 4) * L, L)] = idx_vec

                @pl.when(bmin < BIG)
                def _update():
                    is_syn = bmin == NCOL
                    ci = jnp.where(is_syn, 0, bmin)
                    col = lane(so_v[pl.ds((ci >> 4) * L, L)], ci & (L - 1))
                    w_p = jnp.where(is_syn, W, lax.shift_right_logical(col, 5))
                    j_p = jnp.where(is_syn, 0, col & 31)
                    j_pv = jnp.full((L,), j_p, jnp.int32)
                    bws = [jnp.full((L,), roww[w]) for w in range(W)]
                    bws.append(jnp.full((L,), sw))

                    def upd(tc, _):
                        base = tc * L
                        cb = lax.shift_right_logical(
                            M[w_p, pl.ds(base, L)], j_pv) & 1
                        own = (base + iota) == r
                        negc = jnp.where(own, 0, -cb)
                        for w in range(NWORDS):
                            sl = M[w, pl.ds(base, L)]
                            M[w, pl.ds(base, L)] = sl ^ (bws[w] & negc)
                        return 0

                    lax.fori_loop(0, RANKR // L, upd, 0)

                return idx_vec

            lax.fori_loop(0, RANKR, step, jnp.zeros((L,), jnp.int32))

            pltpu.sync_copy(idxv, idx_hbm.at[b])
            pltpu.sync_copy(M.at[W], sol_hbm.at[b])

    return pl.kernel(
        body,
        out_type=[
            jax.ShapeDtypeStruct((64, RANKR), jnp.int32),
            jax.ShapeDtypeStruct((64, RANKR), jnp.int32),
        ],
        mesh=mesh,
        compiler_params=pltpu.CompilerParams(
            use_tc_tiling_on_sc=False, needs_layout_passes=False),
        scratch_types=[
            pltpu.VMEM((NWORDS, RANKR), jnp.int32),
            pltpu.VMEM((32, L), jnp.int32),
            pltpu.VMEM((NCOL,), jnp.int32),
            pltpu.VMEM((RANKR,), jnp.int32),
        ],
    )


def kernel(llr, pcm, s, bs):
    bs_static = llr.shape[0]
    sort_order = jnp.argsort(llr, axis=-1).astype(jnp.int32)
    inv_sort = jnp.argsort(sort_order, axis=-1).astype(jnp.int32)

    pcm = pcm.astype(jnp.int32)
    shifts = jnp.arange(32, dtype=jnp.int32)
    packed = jnp.sum(
        jnp.left_shift(pcm.reshape(bs_static, RANKR, W, 32), shifts),
        axis=-1, dtype=jnp.int32)
    syn = jnp.transpose(s, (1, 0)).astype(jnp.int32)
    words = jnp.concatenate([packed, syn[:, :, None]], axis=-1)
    words = jnp.transpose(words, (0, 2, 1))

    rankt = jnp.transpose(inv_sort.reshape(bs_static, W, 32), (0, 2, 1))

    info = plsc.get_sparse_core_info()
    elim = _build_elim(info.num_cores, info.num_subcores)
    idx_pivot, sol = elim(words, rankt, sort_order)

    rows = jnp.arange(bs_static)[:, None]
    e_hat = jnp.zeros(llr.shape, dtype=jnp.bool_)
    e_hat = e_hat.at[rows, idx_pivot].set(sol.astype(jnp.bool_), mode="drop")
    e_hat = jnp.take_along_axis(e_hat, inv_sort, axis=-1)
    return e_hat


# speedup vs baseline: 70.5983x; 1.5528x over previous
import functools

import jax
import jax.numpy as jnp
from jax import lax
from jax.experimental import pallas as pl
from jax.experimental.pallas import tpu as pltpu
from jax.experimental.pallas import tpu_sc as plsc

L = 16
W = 16
NWORDS = W + 1
RANKR = 256
NCOL = 512
BIG = 1 << 20


def _build_elim(num_cores, num_subcores):
    nworkers = num_cores * num_subcores
    assert 64 % nworkers == 0
    bpw = 64 // nworkers

    mesh = plsc.VectorSubcoreMesh(core_axis_name="c", subcore_axis_name="s")

    def body(words_hbm, rankt_hbm, so_hbm, ehat_hbm, M, rankT, so_v, idxv, ehat_v):
        wid = lax.axis_index("s") * num_cores + lax.axis_index("c")
        iota = lax.iota(jnp.int32, L)
        zeros = jnp.zeros((L,), jnp.int32)
        bigv = jnp.full((L,), BIG, jnp.int32)

        def lane(vec, i):
            return jnp.sum(jnp.where(iota == i, vec, 0))

        for t in range(bpw):
            b = wid * bpw + t
            pltpu.sync_copy(words_hbm.at[b], M)
            pltpu.sync_copy(rankt_hbm.at[b], rankT)
            pltpu.sync_copy(so_hbm.at[b], so_v)

            def step(r, idx_vec):
                roww = plsc.load_gather(M, [iota, jnp.full((L,), r, jnp.int32)])

                best = bigv
                for j in range(32):
                    bit = lax.shift_right_logical(roww, j) & 1
                    best = jnp.minimum(
                        best, jnp.where(bit == 1, rankT[j], bigv))
                bmin = jnp.min(best)
                sw = lane(M[W, pl.ds((r >> 4) * L, L)], r & (L - 1))
                bmin = jnp.where(sw != 0, jnp.minimum(bmin, NCOL), bmin)

                piv = jnp.where(bmin >= BIG, 0, bmin).astype(jnp.int32)
                idx_vec = jnp.where(iota == (r & (L - 1)), piv, idx_vec)

                @pl.when((r & (L - 1)) == (L - 1))
                def _store_idx():
                    idxv[pl.ds((r >> 4) * L, L)] = idx_vec

                @pl.when(bmin < BIG)
                def _update():
                    is_syn = bmin == NCOL
                    ci = jnp.where(is_syn, 0, bmin)
                    col = lane(so_v[pl.ds((ci >> 4) * L, L)], ci & (L - 1))
                    w_p = jnp.where(is_syn, W, lax.shift_right_logical(col, 5))
                    j_pv = jnp.full((L,), jnp.where(is_syn, 0, col & 31))
                    bws = [jnp.full((L,), roww[w]) for w in range(W)]
                    bws.append(jnp.full((L,), sw))

                    for tc in range(RANKR // L):
                        base = tc * L
                        cb = lax.shift_right_logical(
                            M[w_p, pl.ds(base, L)], j_pv) & 1
                        negc = jnp.where((base + iota) == r, 0, -cb)
                        for w in range(NWORDS):
                            sl = M[w, pl.ds(base, L)]
                            M[w, pl.ds(base, L)] = sl ^ (bws[w] & negc)

                return idx_vec

            lax.fori_loop(0, RANKR, step, zeros)

            for tc in range(NCOL // L):
                ehat_v[pl.ds(tc * L, L)] = zeros
            for tc in range(RANKR // L):
                piv = idxv[pl.ds(tc * L, L)]
                valid = piv < NCOL
                cols = plsc.load_gather(
                    so_v, [jnp.where(valid, piv, 0)])
                solw = M[W, pl.ds(tc * L, L)] & 1
                plsc.store_scatter(ehat_v, [cols], solw, mask=valid)

            pltpu.sync_copy(ehat_v, ehat_hbm.at[b])

    return pl.kernel(
        body,
        out_type=jax.ShapeDtypeStruct((64, NCOL), jnp.int32),
        mesh=mesh,
        compiler_params=pltpu.CompilerParams(
            use_tc_tiling_on_sc=False, needs_layout_passes=False),
        scratch_types=[
            pltpu.VMEM((NWORDS, RANKR), jnp.int32),
            pltpu.VMEM((32, L), jnp.int32),
            pltpu.VMEM((NCOL,), jnp.int32),
            pltpu.VMEM((RANKR,), jnp.int32),
            pltpu.VMEM((NCOL,), jnp.int32),
        ],
    )


def kernel(llr, pcm, s, bs):
    bs_static = llr.shape[0]
    sort_order = jnp.argsort(llr, axis=-1).astype(jnp.int32)
    inv_sort = jnp.argsort(sort_order, axis=-1).astype(jnp.int32)

    pcm = pcm.astype(jnp.int32)
    shifts = jnp.arange(32, dtype=jnp.int32)
    packed = jnp.sum(
        jnp.left_shift(pcm.reshape(bs_static, RANKR, W, 32), shifts),
        axis=-1, dtype=jnp.int32)
    syn = jnp.transpose(s, (1, 0)).astype(jnp.int32)
    words = jnp.concatenate([packed, syn[:, :, None]], axis=-1)
    words = jnp.transpose(words, (0, 2, 1))

    rankt = jnp.transpose(inv_sort.reshape(bs_static, W, 32), (0, 2, 1))

    info = plsc.get_sparse_core_info()
    elim = _build_elim(info.num_cores, info.num_subcores)
    ehat = elim(words, rankt, sort_order)
    return ehat.astype(jnp.bool_)


# speedup vs baseline: 74.4261x; 1.0542x over previous
import functools

import jax
import jax.numpy as jnp
from jax import lax
from jax.experimental import pallas as pl
from jax.experimental.pallas import tpu as pltpu
from jax.experimental.pallas import tpu_sc as plsc

L = 16
W = 16
NWORDS = W + 1
RANKR = 256
NCOL = 512
BIG = 1 << 20
MSIZE = NWORDS * RANKR


def _build_elim(num_cores, num_subcores):
    nworkers = num_cores * num_subcores
    assert 64 % nworkers == 0
    bpw = 64 // nworkers

    mesh = plsc.VectorSubcoreMesh(core_axis_name="c", subcore_axis_name="s")

    def body(words_hbm, rankt_hbm, so_hbm, ehat_hbm, M, rankT, so_v, idxv, ehat_v):
        wid = lax.axis_index("s") * num_cores + lax.axis_index("c")
        iota = lax.iota(jnp.int32, L)
        zeros = jnp.zeros((L,), jnp.int32)
        bigv = jnp.full((L,), BIG, jnp.int32)
        row_stride = iota * RANKR

        for t in range(bpw):
            b = wid * bpw + t
            pltpu.sync_copy(words_hbm.at[pl.ds(b * MSIZE, MSIZE)], M)
            pltpu.sync_copy(rankt_hbm.at[pl.ds(b * NCOL, NCOL)], rankT)
            pltpu.sync_copy(so_hbm.at[pl.ds(b * NCOL, NCOL)], so_v)

            def step(r, idx_vec):
                roww = plsc.load_gather(M, [row_stride + r])
                sw = plsc.load_gather(
                    M, [jnp.full((L,), W * RANKR + r, jnp.int32)])[0]

                acc = [bigv, bigv, bigv, bigv]
                for j in range(32):
                    hit = lax.shift_left(roww, 31 - j) < 0
                    acc[j & 3] = jnp.minimum(
                        acc[j & 3],
                        jnp.where(hit, rankT[pl.ds(j * L, L)], bigv))
                best = jnp.minimum(jnp.minimum(acc[0], acc[1]),
                                   jnp.minimum(acc[2], acc[3]))
                bmin = jnp.min(best)
                bmin = jnp.where(sw != 0, jnp.minimum(bmin, NCOL), bmin)

                piv = jnp.where(bmin >= BIG, 0, bmin).astype(jnp.int32)
                idx_vec = jnp.where(iota == (r & (L - 1)), piv, idx_vec)
                idxv[pl.ds((r >> 4) * L, L)] = idx_vec

                is_syn = bmin >= NCOL
                ci = jnp.where(is_syn, 0, bmin)
                col = plsc.load_gather(so_v, [jnp.full((L,), ci, jnp.int32)])[0]
                w_p = jnp.where(is_syn, W, lax.shift_right_logical(col, 5))
                sh31 = jnp.full((L,), jnp.where(is_syn, 31, 31 - (col & 31)))
                pbase = w_p * RANKR
                bws = [jnp.full((L,), roww[w]) for w in range(W)]
                bws.append(jnp.full((L,), sw))

                for tc in range(RANKR // L):
                    base = tc * L
                    negc = lax.shift_right_arithmetic(
                        lax.shift_left(M[pl.ds(pbase + base, L)], sh31), 31)
                    negc = jnp.where((base + iota) == r, 0, negc)
                    for w in range(NWORDS):
                        sl = M[pl.ds(w * RANKR + base, L)]
                        M[pl.ds(w * RANKR + base, L)] = sl ^ (bws[w] & negc)

                return idx_vec

            lax.fori_loop(0, RANKR, step, zeros)

            for tc in range(NCOL // L):
                ehat_v[pl.ds(tc * L, L)] = zeros
            for tc in range(RANKR // L):
                piv = idxv[pl.ds(tc * L, L)]
                valid = piv < NCOL
                cols = plsc.load_gather(so_v, [jnp.where(valid, piv, 0)])
                solw = M[pl.ds(W * RANKR + tc * L, L)] & 1
                plsc.store_scatter(ehat_v, [cols], solw, mask=valid)

            pltpu.sync_copy(ehat_v, ehat_hbm.at[pl.ds(b * NCOL, NCOL)])

    return pl.kernel(
        body,
        out_type=jax.ShapeDtypeStruct((64 * NCOL,), jnp.int32),
        mesh=mesh,
        compiler_params=pltpu.CompilerParams(
            use_tc_tiling_on_sc=False, needs_layout_passes=False),
        scratch_types=[
            pltpu.VMEM((MSIZE,), jnp.int32),
            pltpu.VMEM((NCOL,), jnp.int32),
            pltpu.VMEM((NCOL,), jnp.int32),
            pltpu.VMEM((RANKR,), jnp.int32),
            pltpu.VMEM((NCOL,), jnp.int32),
        ],
    )


def kernel(llr, pcm, s, bs):
    bs_static = llr.shape[0]
    sort_order = jnp.argsort(llr, axis=-1).astype(jnp.int32)
    inv_sort = jnp.argsort(sort_order, axis=-1).astype(jnp.int32)

    pcm = pcm.astype(jnp.int32)
    shifts = jnp.arange(32, dtype=jnp.int32)
    packed = jnp.sum(
        jnp.left_shift(pcm.reshape(bs_static, RANKR, W, 32), shifts),
        axis=-1, dtype=jnp.int32)
    syn = jnp.transpose(s, (1, 0)).astype(jnp.int32)
    words = jnp.concatenate([packed, syn[:, :, None]], axis=-1)
    words = jnp.transpose(words, (0, 2, 1))

    rankt = jnp.transpose(inv_sort.reshape(bs_static, W, 32), (0, 2, 1))

    info = plsc.get_sparse_core_info()
    elim = _build_elim(info.num_cores, info.num_subcores)
    ehat = elim(words.reshape(-1), rankt.reshape(-1), sort_order.reshape(-1))
    return ehat.reshape(bs_static, NCOL).astype(jnp.bool_)
